# Initial kernel scaffold; baseline (speedup 1.0000x reference)
import jax
import jax.numpy as jnp
from jax.experimental import pallas as pl


def kernel(x, edge_index, W1l, W1r, b1, W2l, W2r, b2):
    raise NotImplementedError("write your pallas kernel here")


# speedup vs baseline: 8.6666x; 8.6666x over previous
import functools

import jax
import jax.numpy as jnp
from jax import lax
from jax.experimental import pallas as pl
from jax.experimental.pallas import tpu as pltpu
from jax.experimental.pallas import tpu_sc as plsc

_N = 10000
_E = 320000
_D = 128
_H = 128
_C = 64

_NC = 2
_NS = 16
_NW = _NC * _NS
_EP = _E // _NW
_K = 80
_NCHUNK = _EP // _K
_NZ = _N // _K
_ZPT = -(-_NZ // _NS)


def _make_sc_agg(df, with_deg):
  mesh = plsc.VectorSubcoreMesh(core_axis_name="c", subcore_axis_name="s")
  out_type = [jax.ShapeDtypeStruct((_NC, _N, df), jnp.float32)]
  scratch = [
      pltpu.VMEM((_NCHUNK, _K), jnp.int32),
      pltpu.VMEM((_NCHUNK, _K), jnp.int32),
      pltpu.VMEM((_K, df), jnp.float32),
      pltpu.VMEM_SHARED((_N, df), jnp.float32),
      pltpu.SemaphoreType.DMA,
  ]
  if with_deg:
    out_type.append(jax.ShapeDtypeStruct((_NC * _N,), jnp.float32))
    scratch += [
        pltpu.VMEM((_K,), jnp.float32),
        pltpu.VMEM((_K,), jnp.float32),
        pltpu.VMEM_SHARED((_N,), jnp.float32),
    ]

  def body(feat_hbm, src_hbm, dst_hbm, agg_out, *rest):
    if with_deg:
      (deg_out, src_v, dst_v, rows_v, agg_sh, sem, ones_v, zeros_v,
       deg_sh) = rest
    else:
      src_v, dst_v, rows_v, agg_sh, sem = rest
      deg_out = ones_v = zeros_v = deg_sh = None

    c = lax.axis_index("c")
    s = lax.axis_index("s")
    wid = c * _NS + s

    zvec = jnp.zeros((16,), jnp.float32)

    def zrow(r, carry):
      for g in range(df // 16):
        rows_v[r, pl.ds(g * 16, 16)] = zvec
      return carry
    lax.fori_loop(0, _K, zrow, 0)
    if with_deg:
      for g in range(_K // 16):
        ones_v[pl.ds(g * 16, 16)] = jnp.ones((16,), jnp.float32)
        zeros_v[pl.ds(g * 16, 16)] = zvec

    for jj in range(_ZPT):
      j = s * _ZPT + jj

      @pl.when(j < _NZ)
      def _():
        pltpu.sync_copy(rows_v, agg_sh.at[pl.ds(j * _K, _K)])
        if with_deg:
          pltpu.sync_copy(zeros_v, deg_sh.at[pl.ds(j * _K, _K)])

    plsc.subcore_barrier()

    pltpu.sync_copy(src_hbm.at[wid], src_v)
    pltpu.sync_copy(dst_hbm.at[wid], dst_v)

    def ebody(j, carry):
      pltpu.async_copy(feat_hbm.at[src_v.at[j]], rows_v, sem).wait()
      pltpu.sync_copy(rows_v, agg_sh.at[dst_v.at[j]], add=True)
      if with_deg:
        pltpu.sync_copy(ones_v, deg_sh.at[dst_v.at[j]], add=True)
      return carry
    lax.fori_loop(0, _NCHUNK, ebody, 0)

    plsc.subcore_barrier()

    for jj in range(_ZPT):
      j = s * _ZPT + jj

      @pl.when(j < _NZ)
      def _():
        pltpu.sync_copy(agg_sh.at[pl.ds(j * _K, _K)],
                        agg_out.at[c, pl.ds(j * _K, _K)])
        if with_deg:
          pltpu.sync_copy(deg_sh.at[pl.ds(j * _K, _K)], zeros_v)
          pltpu.sync_copy(zeros_v, deg_out.at[pl.ds(c * _N + j * _K, _K)])

  return pl.kernel(
      body, out_type=out_type, mesh=mesh, scratch_types=scratch,
      compiler_params=pltpu.CompilerParams(use_tc_tiling_on_sc=False))


_sc_agg_deg = _make_sc_agg(_D, True)
_sc_agg = _make_sc_agg(_C, False)

_BM = 1000


def _tc1_body(agg_ref, degt_ref, x_ref, w1l_ref, w1r_ref, b1_ref,
              w2l_ref, w2r_ref, b2_ref, p_ref, q_ref):
  agg = agg_ref[0] + agg_ref[1]
  degt = degt_ref[...]
  deg = jnp.maximum(degt[:, 0] + degt[:, 1], 1.0)
  mean = agg / deg[:, None]
  h = mean @ w1l_ref[...] + x_ref[...] @ w1r_ref[...] + b1_ref[...]
  h = jnp.maximum(h, 0.0)
  p_ref[...] = h @ w2l_ref[...]
  q_ref[...] = h @ w2r_ref[...] + b2_ref[...]


def _tc2_body(agg_ref, degt_ref, q_ref, out_ref):
  agg = agg_ref[0] + agg_ref[1]
  degt = degt_ref[...]
  deg = jnp.maximum(degt[:, 0] + degt[:, 1], 1.0)
  out_ref[...] = agg / deg[:, None] + q_ref[...]


_tc1 = pl.pallas_call(
    _tc1_body,
    grid=(_N // _BM,),
    in_specs=[
        pl.BlockSpec((_NC, _BM, _D), lambda i: (0, i, 0)),
        pl.BlockSpec((_BM, _NC), lambda i: (i, 0)),
        pl.BlockSpec((_BM, _D), lambda i: (i, 0)),
        pl.BlockSpec((_D, _H), lambda i: (0, 0)),
        pl.BlockSpec((_D, _H), lambda i: (0, 0)),
        pl.BlockSpec((1, _H), lambda i: (0, 0)),
        pl.BlockSpec((_H, _C), lambda i: (0, 0)),
        pl.BlockSpec((_H, _C), lambda i: (0, 0)),
        pl.BlockSpec((1, _C), lambda i: (0, 0)),
    ],
    out_specs=[
        pl.BlockSpec((_BM, _C), lambda i: (i, 0)),
        pl.BlockSpec((_BM, _C), lambda i: (i, 0)),
    ],
    out_shape=[
        jax.ShapeDtypeStruct((_N, _C), jnp.float32),
        jax.ShapeDtypeStruct((_N, _C), jnp.float32),
    ],
)

_tc2 = pl.pallas_call(
    _tc2_body,
    grid=(_N // _BM,),
    in_specs=[
        pl.BlockSpec((_NC, _BM, _C), lambda i: (0, i, 0)),
        pl.BlockSpec((_BM, _NC), lambda i: (i, 0)),
        pl.BlockSpec((_BM, _C), lambda i: (i, 0)),
    ],
    out_specs=pl.BlockSpec((_BM, _C), lambda i: (i, 0)),
    out_shape=jax.ShapeDtypeStruct((_N, _C), jnp.float32),
)


@jax.jit
def kernel(x, edge_index, W1l, W1r, b1, W2l, W2r, b2):
  ei = edge_index.astype(jnp.int32)
  src = ei[0].reshape(_NW, _NCHUNK, _K)
  dst = ei[1].reshape(_NW, _NCHUNK, _K)

  agg1, deg = _sc_agg_deg(x, src, dst)
  degt = deg.reshape(_NC, _N).T

  p, q = _tc1(agg1, degt, x, W1l, W1r, b1.reshape(1, _H),
              W2l, W2r, b2.reshape(1, _C))

  agg2, = _sc_agg(p, src, dst)
  out = _tc2(agg2, degt, q)
  return out
